# baseline (device time: 79155 ns/iter reference)
import jax
import jax.numpy as jnp
from jax import lax
from jax.experimental import pallas as pl
from jax.experimental.pallas import tpu as pltpu

N_DEV = 4
EPS = 1e-5
RB = 512


def kernel(x, gamma, beta):
    M, Nl = x.shape
    nblk = M // RB
    n_global = Nl * N_DEV

    def body(x_ref, g_ref, b_ref, o_ref, xcache, stats, comm, mr, mcols,
             send_sems, recv_sems):
        g = pl.program_id(0)
        my = lax.axis_index("i")

        ones_row = jnp.ones((1, Nl), dtype=jnp.bfloat16)

        @pl.when(g < nblk)
        def _phase0():
            xcb = x_ref[:, :].astype(jnp.bfloat16)
            xcache[pl.ds(g * RB, RB), :] = xcb
            s_row = lax.dot_general(
                ones_row, xcb, (((1,), (1,)), ((), ())),
                preferred_element_type=jnp.float32,
            )
            q_row = lax.dot_general(
                ones_row, xcb * xcb, (((1,), (1,)), ((), ())),
                preferred_element_type=jnp.float32,
            )
            stats[pl.ds(g, 1), :] = s_row
            stats[pl.ds(nblk + g, 1), :] = q_row

        @pl.when(g == nblk)
        def _allreduce():
            barrier_sem = pltpu.get_barrier_semaphore()
            for off in (1, 2, 3):
                pl.semaphore_signal(
                    barrier_sem, inc=1,
                    device_id=(lax.rem(my + off, N_DEV),),
                    device_id_type=pl.DeviceIdType.MESH,
                )
            pl.semaphore_wait(barrier_sem, 3)

            sends = []
            for off in (1, 2, 3):
                r = pltpu.make_async_remote_copy(
                    src_ref=stats,
                    dst_ref=comm.at[my],
                    send_sem=send_sems.at[off],
                    recv_sem=recv_sems.at[my],
                    device_id=(lax.rem(my + off, N_DEV),),
                    device_id_type=pl.DeviceIdType.MESH,
                )
                r.start()
                sends.append(r)

            acc = stats[:, :]
            for off in (1, 2, 3):
                src = lax.rem(my - off + N_DEV, N_DEV)
                recv = pltpu.make_async_remote_copy(
                    src_ref=stats,
                    dst_ref=comm.at[src],
                    send_sem=send_sems.at[0],
                    recv_sem=recv_sems.at[src],
                    device_id=(my,),
                    device_id_type=pl.DeviceIdType.MESH,
                )
                recv.wait_recv()
                acc = acc + comm[src]

            mean_rows = acc[:nblk, :] / n_global
            var_rows = acc[nblk:, :] / n_global - mean_rows * mean_rows
            rstd_rows = lax.rsqrt(var_rows + EPS)
            mr[pl.ds(0, nblk), :] = mean_rows
            mr[pl.ds(nblk, nblk), :] = rstd_rows

            eye = jnp.eye(RB, dtype=jnp.float32)
            cols = lax.dot_general(
                eye, mr[:, :], (((1,), (1,)), ((), ())),
                preferred_element_type=jnp.float32,
            )
            for j in range(2 * nblk):
                mcols[j, :, :] = cols[:, j:j + 1]

            for r in sends:
                r.wait_send()

        @pl.when(g >= nblk)
        def _phase1():
            b = g - nblk
            m_col = mcols[b]
            r_col = mcols[nblk + b]
            xb = xcache[pl.ds(b * RB, RB), :].astype(jnp.float32)
            o_ref[:, :] = (
                (xb - m_col) * r_col * g_ref[:, :] + b_ref[:, :]
            ).astype(jnp.bfloat16)

    out = pl.pallas_call(
        body,
        grid=(2 * nblk,),
        in_specs=[
            pl.BlockSpec((RB, Nl), lambda g: (jnp.minimum(g, nblk - 1), 0)),
            pl.BlockSpec((1, Nl), lambda g: (0, 0)),
            pl.BlockSpec((1, Nl), lambda g: (0, 0)),
        ],
        out_specs=pl.BlockSpec((RB, Nl), lambda g: (jnp.maximum(g - nblk, 0), 0)),
        out_shape=jax.ShapeDtypeStruct((M, Nl), jnp.bfloat16),
        scratch_shapes=[
            pltpu.VMEM((M, Nl), jnp.bfloat16),
            pltpu.VMEM((2 * nblk, RB), jnp.float32),
            pltpu.VMEM((N_DEV, 2 * nblk, RB), jnp.float32),
            pltpu.VMEM((2 * nblk, RB), jnp.float32),
            pltpu.VMEM((2 * nblk, RB, 1), jnp.float32),
            pltpu.SemaphoreType.DMA((N_DEV,)),
            pltpu.SemaphoreType.DMA((N_DEV,)),
        ],
        compiler_params=pltpu.CompilerParams(
            collective_id=0,
            vmem_limit_bytes=100 * 1024 * 1024,
        ),
    )(x, gamma.reshape(1, Nl), beta.reshape(1, Nl))
    return out
